# baseline (speedup 1.0000x reference)
import functools

import jax
import jax.numpy as jnp
from jax import lax
from jax.experimental import pallas as pl
from jax.experimental.pallas import tpu as pltpu
from jax.experimental.pallas import tpu_sc as plsc

B = 16
A = 3
C10 = 10
G = 152
CELLS = G * G
STRIDE = 4.0
NQ = B * A * CELLS

CHUNK = 5776
CHUNKS_PER_PAIR = CELLS // CHUNK
ROWS_PER_CHUNK = CHUNK // G
N_CHUNKS = B * A * CHUNKS_PER_PAIR
N_WORKERS = 32
CHUNKS_PER_WORKER = N_CHUNKS // N_WORKERS
GROUPS = CHUNK // 16

ANC_W = (12.0, 19.0, 40.0)
ANC_H = (16.0, 36.0, 28.0)

_RCP_MAGIC = 0x7EF311C3


def _rcp(d):
    i = plsc.bitcast(d, jnp.int32)
    x = plsc.bitcast(_RCP_MAGIC - i, jnp.float32)
    x = x * (2.0 - d * x)
    x = x * (2.0 - d * x)
    x = x * (2.0 - d * x)
    return x


def _sigmoid(v):
    e = jnp.exp(-jnp.maximum(v, -30.0))
    return _rcp(1.0 + e)


def _yolo_body(x_hbm, out_hbm, in_v, out_v):
    nb = x_hbm.shape[0]
    chunks_per_worker = nb * A * CHUNKS_PER_PAIR // N_WORKERS
    w = lax.axis_index("s") * 2 + lax.axis_index("c")
    lane = lax.broadcasted_iota(jnp.int32, (16,), 0)

    def chunk_body(t, carry):
        cid = w * chunks_per_worker + t
        pair = cid // CHUNKS_PER_PAIR
        j = cid - pair * CHUNKS_PER_PAIR
        b = pair // A
        a = pair - b * A
        off = j * CHUNK

        pltpu.sync_copy(
            x_hbm.at[b, pl.ds(a * C10, C10), pl.ds(off, CHUNK)], in_v)

        aw = jnp.where(a == 0, ANC_W[0], jnp.where(a == 1, ANC_W[1], ANC_W[2]))
        ah = jnp.where(a == 0, ANC_H[0], jnp.where(a == 1, ANC_H[1], ANC_H[2]))

        gx0 = lane.astype(jnp.float32)
        gy0 = jnp.zeros((16,), jnp.float32) + (ROWS_PER_CHUNK * j).astype(
            jnp.float32)

        def group_body(g, carry2):
            gxf, gyf, base = carry2

            p0 = in_v[0, pl.ds(base, 16)]
            p1 = in_v[1, pl.ds(base, 16)]
            p2 = in_v[2, pl.ds(base, 16)]
            p3 = in_v[3, pl.ds(base, 16)]
            p4 = in_v[4, pl.ds(base, 16)]
            p5 = in_v[5, pl.ds(base, 16)]
            p6 = in_v[6, pl.ds(base, 16)]
            p7 = in_v[7, pl.ds(base, 16)]
            p8 = in_v[8, pl.ds(base, 16)]
            p9 = in_v[9, pl.ds(base, 16)]

            out_v[0, pl.ds(base, 16)] = (_sigmoid(p0) + gxf) * STRIDE
            out_v[1, pl.ds(base, 16)] = (_sigmoid(p1) + gyf) * STRIDE
            out_v[2, pl.ds(base, 16)] = jnp.exp(p2) * aw
            out_v[3, pl.ds(base, 16)] = jnp.exp(p3) * ah
            out_v[4, pl.ds(base, 16)] = p4
            out_v[5, pl.ds(base, 16)] = p5
            out_v[6, pl.ds(base, 16)] = _sigmoid(p6)
            out_v[7, pl.ds(base, 16)] = _sigmoid(p7)
            out_v[8, pl.ds(base, 16)] = _sigmoid(p8)
            out_v[9, pl.ds(base, 16)] = _sigmoid(p9)

            gx2 = gxf + 16.0
            wrap = gx2 >= float(G)
            gxn = jnp.where(wrap, gx2 - float(G), gx2)
            gyn = jnp.where(wrap, gyf + 1.0, gyf)
            return (gxn, gyn, base + 16)

        lax.fori_loop(0, GROUPS, group_body, (gx0, gy0, 0))

        dst = pair * CELLS + off
        pltpu.sync_copy(out_v, out_hbm.at[:, pl.ds(dst, CHUNK)])
        return carry

    lax.fori_loop(0, chunks_per_worker, chunk_body, 0)


def _make_call(nb):
    return functools.partial(
        pl.kernel,
        out_type=jax.ShapeDtypeStruct((C10, nb * A * CELLS), jnp.float32),
        mesh=plsc.VectorSubcoreMesh(
            core_axis_name="c", subcore_axis_name="s",
            num_cores=2, num_subcores=16),
        scratch_types=[
            pltpu.VMEM((C10, CHUNK), jnp.float32),
            pltpu.VMEM((C10, CHUNK), jnp.float32),
        ],
        compiler_params=pltpu.CompilerParams(
            use_tc_tiling_on_sc=False, needs_layout_passes=False),
    )(_yolo_body)


BH = 8
_yolo_half = _make_call(BH)


def kernel(x):
    xa = x[:BH].reshape(BH, A * C10, CELLS)
    xb = x[BH:].reshape(BH, A * C10, CELLS)
    ya = _yolo_half(xa).reshape(C10, BH, A * CELLS)
    yb = _yolo_half(xb).reshape(C10, BH, A * CELLS)
    planar = jnp.concatenate([ya, yb], axis=1)
    return planar.transpose(1, 2, 0)


# speedup vs baseline: 1.0854x; 1.0854x over previous
import functools

import jax
import jax.numpy as jnp
from jax import lax
from jax.experimental import pallas as pl
from jax.experimental.pallas import tpu as pltpu
from jax.experimental.pallas import tpu_sc as plsc

B = 16
A = 3
C10 = 10
G = 152
CELLS = G * G
STRIDE = 4.0
NQ = B * A * CELLS

CHUNK = 5776
CHUNKS_PER_PAIR = CELLS // CHUNK
ROWS_PER_CHUNK = CHUNK // G
N_CHUNKS = B * A * CHUNKS_PER_PAIR
N_WORKERS = 32
CHUNKS_PER_WORKER = N_CHUNKS // N_WORKERS
GROUPS = CHUNK // 16

ANC_W = (12.0, 19.0, 40.0)
ANC_H = (16.0, 36.0, 28.0)

_RCP_MAGIC = 0x7EF311C3


def _rcp(d):
    i = plsc.bitcast(d, jnp.int32)
    x = plsc.bitcast(_RCP_MAGIC - i, jnp.float32)
    x = x * (2.0 - d * x)
    x = x * (2.0 - d * x)
    x = x * (2.0 - d * x)
    return x


def _sigmoid(v):
    e = jnp.exp(-jnp.maximum(v, -30.0))
    return _rcp(1.0 + e)


def _yolo_body(x_hbm, out_hbm, in_v, out_v):
    w = lax.axis_index("s") * 2 + lax.axis_index("c")
    lane = lax.broadcasted_iota(jnp.int32, (16,), 0)

    def chunk_body(t, carry):
        cid = w * CHUNKS_PER_WORKER + t
        pair = cid // CHUNKS_PER_PAIR
        j = cid - pair * CHUNKS_PER_PAIR
        b = pair // A
        a = pair - b * A
        off = j * CHUNK

        pltpu.sync_copy(
            x_hbm.at[b, pl.ds(a * C10, C10), pl.ds(off, CHUNK)], in_v)

        aw = jnp.where(a == 0, ANC_W[0], jnp.where(a == 1, ANC_W[1], ANC_W[2]))
        ah = jnp.where(a == 0, ANC_H[0], jnp.where(a == 1, ANC_H[1], ANC_H[2]))

        gx0 = lane.astype(jnp.float32)
        gy0 = jnp.zeros((16,), jnp.float32) + (ROWS_PER_CHUNK * j).astype(
            jnp.float32)

        def group_body(g, carry2):
            gxf, gyf, base = carry2

            p0 = in_v[0, pl.ds(base, 16)]
            p1 = in_v[1, pl.ds(base, 16)]
            p2 = in_v[2, pl.ds(base, 16)]
            p3 = in_v[3, pl.ds(base, 16)]
            p4 = in_v[4, pl.ds(base, 16)]
            p5 = in_v[5, pl.ds(base, 16)]
            p6 = in_v[6, pl.ds(base, 16)]
            p7 = in_v[7, pl.ds(base, 16)]
            p8 = in_v[8, pl.ds(base, 16)]
            p9 = in_v[9, pl.ds(base, 16)]

            out_v[0, pl.ds(base, 16)] = (_sigmoid(p0) + gxf) * STRIDE
            out_v[1, pl.ds(base, 16)] = (_sigmoid(p1) + gyf) * STRIDE
            out_v[2, pl.ds(base, 16)] = jnp.exp(p2) * aw
            out_v[3, pl.ds(base, 16)] = jnp.exp(p3) * ah
            out_v[4, pl.ds(base, 16)] = p4
            out_v[5, pl.ds(base, 16)] = p5
            out_v[6, pl.ds(base, 16)] = _sigmoid(p6)
            out_v[7, pl.ds(base, 16)] = _sigmoid(p7)
            out_v[8, pl.ds(base, 16)] = _sigmoid(p8)
            out_v[9, pl.ds(base, 16)] = _sigmoid(p9)

            gx2 = gxf + 16.0
            wrap = gx2 >= float(G)
            gxn = jnp.where(wrap, gx2 - float(G), gx2)
            gyn = jnp.where(wrap, gyf + 1.0, gyf)
            return (gxn, gyn, base + 16)

        lax.fori_loop(0, GROUPS, group_body, (gx0, gy0, 0))

        dst = pair * CELLS + off
        pltpu.sync_copy(out_v, out_hbm.at[:, pl.ds(dst, CHUNK)])
        return carry

    lax.fori_loop(0, CHUNKS_PER_WORKER, chunk_body, 0)


_yolo_sc = functools.partial(
    pl.kernel,
    out_type=jax.ShapeDtypeStruct((C10, NQ), jnp.float32),
    mesh=plsc.VectorSubcoreMesh(
        core_axis_name="c", subcore_axis_name="s",
        num_cores=2, num_subcores=16),
    scratch_types=[
        pltpu.VMEM((C10, CHUNK), jnp.float32),
        pltpu.VMEM((C10, CHUNK), jnp.float32),
    ],
    compiler_params=pltpu.CompilerParams(
        use_tc_tiling_on_sc=False, needs_layout_passes=False),
)(_yolo_body)


def kernel(x):
    x3 = x.reshape(B, A * C10, CELLS)
    planar = _yolo_sc(x3)
    return planar.reshape(C10, B, A * CELLS).transpose(1, 2, 0)
